# baseline (speedup 1.0000x reference)
import jax
import jax.numpy as jnp
from jax import lax
from jax.experimental import pallas as pl
from jax.experimental.pallas import tpu as pltpu
from jax.experimental.pallas import tpu_sc as plsc

D = 64
B = 1024
S = 200
NT = 20
SG = S - NT
NC = 2
NS = 16
NW = NC * NS
BPW = B // NW
CH = 90
NCH = SG // CH


def _sc_body(idx_hbm, wte_hbm, learned_hbm, out_hbm, idx_v, rows_v, gsem):
    w = lax.axis_index("s") * NC + lax.axis_index("c")

    pltpu.sync_copy(idx_hbm.at[pl.ds(w * BPW * NCH, BPW * NCH)], idx_v)
    pltpu.sync_copy(learned_hbm, rows_v.at[0, pl.ds(0, NT)])
    pltpu.sync_copy(learned_hbm, rows_v.at[1, pl.ds(0, NT)])

    def one_batch(j, p):
        cp0 = pltpu.async_copy(
            wte_hbm.at[idx_v.at[NCH * j]], rows_v.at[p, pl.ds(NT, CH)], gsem)
        cp1 = pltpu.async_copy(
            wte_hbm.at[idx_v.at[NCH * j + 1]], rows_v.at[p, pl.ds(NT + CH, CH)], gsem)
        cp0.wait()
        cp1.wait()
        pltpu.sync_copy(rows_v.at[p], out_hbm.at[pl.ds((w * BPW + j) * S, S)])

    def body(i, carry):
        one_batch(2 * i, 0)
        one_batch(2 * i + 1, 1)
        return carry

    lax.fori_loop(0, BPW // 2, body, 0)


@jax.jit
def _gather(idx2d, wte_weight, learned_embedding):
    mesh = plsc.VectorSubcoreMesh(core_axis_name="c", subcore_axis_name="s")
    return pl.kernel(
        _sc_body,
        out_type=jax.ShapeDtypeStruct((B * S, D), jnp.float32),
        mesh=mesh,
        scratch_types=[
            pltpu.VMEM((BPW * NCH, CH), jnp.int32),
            pltpu.VMEM((2, S, D), jnp.float32),
            pltpu.SemaphoreType.DMA,
        ],
        compiler_params=pltpu.CompilerParams(use_tc_tiling_on_sc=False),
    )(idx2d, wte_weight, learned_embedding)


def kernel(tokens, wte_weight, learned_embedding):
    idx2d = tokens[:, NT:].reshape(B * NCH, CH)
    out = _gather(idx2d, wte_weight, learned_embedding)
    return out.reshape(B, S, D)


# speedup vs baseline: 1.0253x; 1.0253x over previous
import jax
import jax.numpy as jnp
from jax import lax
from jax.experimental import pallas as pl
from jax.experimental.pallas import tpu as pltpu
from jax.experimental.pallas import tpu_sc as plsc

D = 64
B = 1024
S = 200
NT = 20
SG = S - NT
NC = 2
NS = 16
NW = NC * NS
BPW = B // NW
CH = 90
NCH = SG // CH
NBUF = 6
GDEPTH = 3


def _sc_body(idx_hbm, wte_hbm, learned_hbm, out_hbm, idx_v, rows_v, gsem, osem):
    w = lax.axis_index("s") * NC + lax.axis_index("c")

    def drain_out():
        pltpu.make_async_copy(out_hbm.at[pl.ds(0, S)], rows_v.at[0], osem).wait()

    def drain_gather():
        pltpu.make_async_copy(
            out_hbm.at[pl.ds(0, CH)], rows_v.at[0, pl.ds(NT, CH)], gsem).wait()

    pltpu.sync_copy(idx_hbm.at[pl.ds(w * BPW * NCH, BPW * NCH)], idx_v)
    for k in range(NBUF):
        pltpu.sync_copy(learned_hbm, rows_v.at[k, pl.ds(0, NT)])

    def body(j, carry):
        p = lax.rem(j, NBUF)

        @pl.when(j >= NBUF)
        def _():
            drain_out()

        @pl.when(j < BPW)
        def _():
            pltpu.async_copy(
                wte_hbm.at[idx_v.at[NCH * j]],
                rows_v.at[p, pl.ds(NT, CH)], gsem)
            pltpu.async_copy(
                wte_hbm.at[idx_v.at[NCH * j + 1]],
                rows_v.at[p, pl.ds(NT + CH, CH)], gsem)

        @pl.when(j >= GDEPTH)
        def _():
            t = j - GDEPTH
            q = lax.rem(t, NBUF)
            drain_gather()
            drain_gather()
            pltpu.async_copy(
                rows_v.at[q], out_hbm.at[pl.ds((w * BPW + t) * S, S)], osem)

        return carry

    lax.fori_loop(0, BPW + GDEPTH, body, 0)
    for _ in range(NBUF - GDEPTH):
        drain_out()


@jax.jit
def _gather(idx2d, wte_weight, learned_embedding):
    mesh = plsc.VectorSubcoreMesh(core_axis_name="c", subcore_axis_name="s")
    return pl.kernel(
        _sc_body,
        out_type=jax.ShapeDtypeStruct((B * S, D), jnp.float32),
        mesh=mesh,
        scratch_types=[
            pltpu.VMEM((BPW * NCH, CH), jnp.int32),
            pltpu.VMEM((NBUF, S, D), jnp.float32),
            pltpu.SemaphoreType.DMA,
            pltpu.SemaphoreType.DMA,
        ],
        compiler_params=pltpu.CompilerParams(use_tc_tiling_on_sc=False),
    )(idx2d, wte_weight, learned_embedding)


def kernel(tokens, wte_weight, learned_embedding):
    idx2d = tokens[:, NT:].reshape(B * NCH, CH)
    out = _gather(idx2d, wte_weight, learned_embedding)
    return out.reshape(B, S, D)
